# Initial kernel scaffold; baseline (speedup 1.0000x reference)
import jax
import jax.numpy as jnp
from jax.experimental import pallas as pl


def kernel(x_raw, edge_index, batch, W_l0, b_l0, W_r0, W_l1, b_l1, W_r1):
    raise NotImplementedError("write your pallas kernel here")


# speedup vs baseline: 10.3762x; 10.3762x over previous
import functools

import jax
import jax.numpy as jnp
from jax import lax
from jax.experimental import pallas as pl
from jax.experimental.pallas import tpu as pltpu
from jax.experimental.pallas import tpu_sc as plsc

N = 10000
E = 320000
D = 128
B = 64

NC = 2
NS = 16
NW = NC * NS
EPT = E // NW
CHUNK = 80
NITER = EPT // CHUNK
IBLK = 25
NBLK = NITER // IBLK
NPAIR = (IBLK - 1) // 2
TROWS = 640
CP = 80
NQ = TROWS // CP
CW = 16

BLK = 1000
GRID = N // BLK


def _mesh():
    return plsc.VectorSubcoreMesh(core_axis_name="c", subcore_axis_name="s",
                                  num_cores=NC, num_subcores=NS)


def _own_copies(sid, r0, fn):
    notlast = sid < NS - 1
    for q in range(NQ):
        if q < 5:
            fn(r0 + q * CP)
        else:
            @pl.when(notlast)
            def _():
                fn(r0 + q * CP)


@functools.lru_cache(maxsize=None)
def _make_sc_agg():
    scratch = [
        pltpu.VMEM((IBLK, CHUNK), jnp.int32),
        pltpu.VMEM((IBLK, CHUNK), jnp.int32),
        pltpu.VMEM((CHUNK, D), jnp.float32),
        pltpu.VMEM((CHUNK, D), jnp.float32),
        pltpu.VMEM_SHARED((N, D), jnp.float32),
        pltpu.SemaphoreType.DMA,
        pltpu.SemaphoreType.DMA,
    ]

    def body(x_hbm, src_hbm, dst_hbm, part_hbm,
             src_i, dst_i, rows0, rows1, acc, sem0, sem1):
        cid = lax.axis_index("c")
        sid = lax.axis_index("s")
        wid = cid * NS + sid
        r0 = sid * TROWS

        def zrow(i, c):
            for j in range(D // 16):
                rows0[i, pl.ds(j * 16, 16)] = jnp.zeros((16,), jnp.float32)
            return c
        lax.fori_loop(0, CHUNK, zrow, 0)
        _own_copies(sid, r0,
                    lambda rr: pltpu.sync_copy(rows0, acc.at[pl.ds(rr, CP)]))

        plsc.subcore_barrier()

        def fire(it, buf, sem):
            pltpu.async_copy(x_hbm.at[src_i.at[it]], buf, sem)

        def drain(buf, sem):
            pltpu.make_async_copy(x_hbm.at[src_i.at[0]], buf, sem).wait()

        def scat(it, buf):
            pltpu.sync_copy(buf, acc.at[dst_i.at[it]], add=True)

        for b in range(NBLK):
            pltpu.sync_copy(src_hbm.at[wid, b], src_i)
            pltpu.sync_copy(dst_hbm.at[wid, b], dst_i)
            fire(0, rows0, sem0)

            def step(k, c):
                g = 2 * k
                fire(g + 1, rows1, sem1)
                drain(rows0, sem0)
                scat(g, rows0)
                fire(g + 2, rows0, sem0)
                drain(rows1, sem1)
                scat(g + 1, rows1)
                return c
            lax.fori_loop(0, NPAIR, step, 0)
            drain(rows0, sem0)
            scat(IBLK - 1, rows0)

        plsc.subcore_barrier()

        def cp_out(rr):
            pltpu.sync_copy(acc.at[pl.ds(rr, CP)], rows0)
            pltpu.sync_copy(rows0, part_hbm.at[cid, pl.ds(rr, CP)])
        _own_copies(sid, r0, cp_out)

    return pl.kernel(body,
                     out_type=[jax.ShapeDtypeStruct((NC, N, D), jnp.float32)],
                     mesh=_mesh(), scratch_types=scratch)


@functools.lru_cache(maxsize=None)
def _make_sc_cnt():
    scratch = [
        pltpu.VMEM((IBLK, CHUNK), jnp.int32),
        pltpu.VMEM((N,), jnp.float32),
    ]

    def body(dst_hbm, cntp_hbm, dst_i, cnt_v):
        cid = lax.axis_index("c")
        sid = lax.axis_index("s")
        wid = cid * NS + sid

        def zero(i, c):
            cnt_v[pl.ds(i * 16, 16)] = jnp.zeros((16,), jnp.float32)
            return c
        lax.fori_loop(0, N // 16, zero, 0)

        ones = jnp.full((16,), 1.0, jnp.float32)
        for b in range(NBLK):
            pltpu.sync_copy(dst_hbm.at[wid, b], dst_i)

            def step(i, c):
                for j in range(CHUNK // 16):
                    idx = dst_i[i, pl.ds(j * 16, 16)]
                    plsc.addupdate_scatter(cnt_v, [idx], ones)
                return c
            lax.fori_loop(0, IBLK, step, 0)

        pltpu.sync_copy(cnt_v, cntp_hbm.at[wid])

    return pl.kernel(body,
                     out_type=[jax.ShapeDtypeStruct((NW, N), jnp.float32)],
                     mesh=_mesh(),
                     compiler_params=pltpu.CompilerParams(
                         needs_layout_passes=False),
                     scratch_types=scratch)


def _layer_math(p_ref, x_ref, cnt_ref, wl_ref, wr_ref, b_ref):
    p = p_ref[...]
    agg = p[0] + p[1]
    cnt = jnp.sum(cnt_ref[...], axis=1, keepdims=True)
    mean = agg / jnp.maximum(cnt, 1.0)
    z = (jnp.dot(mean, wl_ref[...], preferred_element_type=jnp.float32,
                 precision=lax.Precision.HIGHEST)
         + jnp.dot(x_ref[...], wr_ref[...], preferred_element_type=jnp.float32,
                   precision=lax.Precision.HIGHEST)
         + b_ref[...])
    nrm = jnp.sqrt(jnp.sum(z * z, axis=1, keepdims=True))
    h = z / jnp.maximum(nrm, 1e-12)
    return jnp.maximum(h, 0.0)


def _tc_layer_body(p_ref, x_ref, cnt_ref, wl_ref, wr_ref, b_ref, o_ref):
    o_ref[...] = _layer_math(p_ref, x_ref, cnt_ref, wl_ref, wr_ref, b_ref)


def _tc_layer2_body(p_ref, x_ref, cnt_ref, wl_ref, wr_ref, b_ref, batch_ref,
                    g_ref):
    h = _layer_math(p_ref, x_ref, cnt_ref, wl_ref, wr_ref, b_ref)
    oh = (batch_ref[...] ==
          lax.broadcasted_iota(jnp.int32, (BLK, B), 1)).astype(jnp.float32)
    contrib = lax.dot_general(oh, h, (((0,), (0,)), ((), ())),
                              preferred_element_type=jnp.float32,
                              precision=lax.Precision.HIGHEST)

    @pl.when(pl.program_id(0) == 0)
    def _():
        g_ref[...] = jnp.zeros_like(g_ref)
    g_ref[...] += contrib


_COMMON_SPECS = [
    pl.BlockSpec((NC, BLK, D), lambda i: (0, i, 0)),
    pl.BlockSpec((BLK, D), lambda i: (i, 0)),
    pl.BlockSpec((BLK, NC * CW), lambda i: (i, 0)),
    pl.BlockSpec((D, D), lambda i: (0, 0)),
    pl.BlockSpec((D, D), lambda i: (0, 0)),
    pl.BlockSpec((1, D), lambda i: (0, 0)),
]


def _tc_layer(p, x, cnt2, wlT, wrT, b2):
    return pl.pallas_call(
        _tc_layer_body,
        grid=(GRID,),
        in_specs=_COMMON_SPECS,
        out_specs=pl.BlockSpec((BLK, D), lambda i: (i, 0)),
        out_shape=jax.ShapeDtypeStruct((N, D), jnp.float32),
    )(p, x, cnt2, wlT, wrT, b2)


def _tc_layer2(p, x, cnt2, wlT, wrT, b2, batch2):
    return pl.pallas_call(
        _tc_layer2_body,
        grid=(GRID,),
        in_specs=_COMMON_SPECS + [pl.BlockSpec((BLK, 1), lambda i: (i, 0))],
        out_specs=pl.BlockSpec((B, D), lambda i: (0, 0)),
        out_shape=jax.ShapeDtypeStruct((B, D), jnp.float32),
    )(p, x, cnt2, wlT, wrT, b2, batch2)


def kernel(x_raw, edge_index, batch, W_l0, b_l0, W_r0, W_l1, b_l1, W_r1):
    src4 = edge_index[0].reshape(NW, NBLK, IBLK, CHUNK)
    dst4 = edge_index[1].reshape(NW, NBLK, IBLK, CHUNK)
    (cntp,) = _make_sc_cnt()(dst4)
    (part0,) = _make_sc_agg()(x_raw, src4, dst4)
    cnt2 = cntp.T
    h0 = _tc_layer(part0, x_raw, cnt2, W_l0.T, W_r0.T, b_l0.reshape(1, D))
    (part1,) = _make_sc_agg()(h0, src4, dst4)
    g = _tc_layer2(part1, h0, cnt2, W_l1.T, W_r1.T, b_l1.reshape(1, D),
                   batch.reshape(N, 1))
    return g


# speedup vs baseline: 10.7377x; 1.0348x over previous
import functools

import jax
import jax.numpy as jnp
from jax import lax
from jax.experimental import pallas as pl
from jax.experimental.pallas import tpu as pltpu
from jax.experimental.pallas import tpu_sc as plsc

N = 10000
E = 320000
D = 128
B = 64

NC = 2
NS = 16
NW = NC * NS
EPT = E // NW
CHUNK = 80
NITER = EPT // CHUNK
IBLK = 25
NBLK = NITER // IBLK
NPAIR = (IBLK - 1) // 2
TROWS = 640
CP = 80
NQ = TROWS // CP
CW = 16

BLK = 1000
GRID = N // BLK


def _mesh():
    return plsc.VectorSubcoreMesh(core_axis_name="c", subcore_axis_name="s",
                                  num_cores=NC, num_subcores=NS)


def _own_copies(sid, r0, fn):
    notlast = sid < NS - 1
    for q in range(NQ):
        if q < 5:
            fn(r0 + q * CP)
        else:
            @pl.when(notlast)
            def _():
                fn(r0 + q * CP)


@functools.lru_cache(maxsize=None)
def _make_sc_agg(with_cnt: bool):
    out_type = [jax.ShapeDtypeStruct((NC, N, D), jnp.float32)]
    scratch = [
        pltpu.VMEM((IBLK, CHUNK), jnp.int32),
        pltpu.VMEM((IBLK, CHUNK), jnp.int32),
        pltpu.VMEM((IBLK, CHUNK), jnp.int32),
        pltpu.VMEM((IBLK, CHUNK), jnp.int32),
        pltpu.VMEM((CHUNK, D), jnp.float32),
        pltpu.VMEM((CHUNK, D), jnp.float32),
        pltpu.VMEM_SHARED((N, D), jnp.float32),
        pltpu.SemaphoreType.DMA,
        pltpu.SemaphoreType.DMA,
        pltpu.SemaphoreType.DMA,
    ]
    if with_cnt:
        out_type.append(jax.ShapeDtypeStruct((NW, N), jnp.float32))
        scratch.append(pltpu.VMEM((N,), jnp.float32))

    def body(*refs):
        if with_cnt:
            (x_hbm, src_hbm, dst_hbm, part_hbm, cntp_hbm,
             src_i0, dst_i0, src_i1, dst_i1, rows0, rows1, acc,
             sem0, sem1, semi, cnt_v) = refs
        else:
            (x_hbm, src_hbm, dst_hbm, part_hbm,
             src_i0, dst_i0, src_i1, dst_i1, rows0, rows1, acc,
             sem0, sem1, semi) = refs
        srcs = [src_i0, src_i1]
        dsts = [dst_i0, dst_i1]

        cid = lax.axis_index("c")
        sid = lax.axis_index("s")
        wid = cid * NS + sid
        r0 = sid * TROWS

        def zrow(i, c):
            for j in range(D // 16):
                rows0[i, pl.ds(j * 16, 16)] = jnp.zeros((16,), jnp.float32)
            return c
        lax.fori_loop(0, CHUNK, zrow, 0)
        _own_copies(sid, r0,
                    lambda rr: pltpu.sync_copy(rows0, acc.at[pl.ds(rr, CP)]))
        if with_cnt:
            def zcnt(i, c):
                cnt_v[pl.ds(i * 16, 16)] = jnp.zeros((16,), jnp.float32)
                return c
            lax.fori_loop(0, N // 16, zcnt, 0)
        ones = jnp.full((16,), 1.0, jnp.float32)

        pltpu.sync_copy(src_hbm.at[wid, 0], srcs[0])
        pltpu.sync_copy(dst_hbm.at[wid, 0], dsts[0])

        plsc.subcore_barrier()

        for b in range(NBLK):
            src_i = srcs[b % 2]
            dst_i = dsts[b % 2]
            if b + 1 < NBLK:
                pltpu.async_copy(src_hbm.at[wid, b + 1], srcs[(b + 1) % 2],
                                 semi)
                pltpu.async_copy(dst_hbm.at[wid, b + 1], dsts[(b + 1) % 2],
                                 semi)

            def fire(it, buf, sem):
                pltpu.async_copy(x_hbm.at[src_i.at[it]], buf, sem)

            def drain(buf, sem):
                pltpu.make_async_copy(x_hbm.at[src_i.at[0]], buf, sem).wait()

            def scat(it, buf):
                pltpu.sync_copy(buf, acc.at[dst_i.at[it]], add=True)
                if with_cnt:
                    for j in range(CHUNK // 16):
                        idx = dst_i[it, pl.ds(j * 16, 16)]
                        plsc.addupdate_scatter(cnt_v, [idx], ones)

            fire(0, rows0, sem0)

            def step(k, c):
                g = 2 * k
                fire(g + 1, rows1, sem1)
                drain(rows0, sem0)
                scat(g, rows0)
                fire(g + 2, rows0, sem0)
                drain(rows1, sem1)
                scat(g + 1, rows1)
                return c
            lax.fori_loop(0, NPAIR, step, 0)
            drain(rows0, sem0)
            scat(IBLK - 1, rows0)

            if b + 1 < NBLK:
                pltpu.make_async_copy(src_hbm.at[wid, b + 1],
                                      srcs[(b + 1) % 2], semi).wait()
                pltpu.make_async_copy(dst_hbm.at[wid, b + 1],
                                      dsts[(b + 1) % 2], semi).wait()

        plsc.subcore_barrier()

        def cp_out(rr):
            pltpu.sync_copy(acc.at[pl.ds(rr, CP)], rows0)
            pltpu.sync_copy(rows0, part_hbm.at[cid, pl.ds(rr, CP)])
        _own_copies(sid, r0, cp_out)
        if with_cnt:
            pltpu.sync_copy(cnt_v, cntp_hbm.at[wid])

    params = pltpu.CompilerParams(needs_layout_passes=False)
    return pl.kernel(body, out_type=out_type, mesh=_mesh(),
                     compiler_params=params, scratch_types=scratch)


def _layer_math(p_ref, x_ref, cnt_ref, wl_ref, wr_ref, b_ref):
    p = p_ref[...]
    agg = p[0] + p[1]
    cnt = jnp.sum(cnt_ref[...], axis=1, keepdims=True)
    mean = agg / jnp.maximum(cnt, 1.0)
    z = (jnp.dot(mean, wl_ref[...], preferred_element_type=jnp.float32,
                 precision=lax.Precision.HIGHEST)
         + jnp.dot(x_ref[...], wr_ref[...], preferred_element_type=jnp.float32,
                   precision=lax.Precision.HIGHEST)
         + b_ref[...])
    nrm = jnp.sqrt(jnp.sum(z * z, axis=1, keepdims=True))
    h = z / jnp.maximum(nrm, 1e-12)
    return jnp.maximum(h, 0.0)


def _tc_layer_body(p_ref, x_ref, cnt_ref, wl_ref, wr_ref, b_ref, o_ref):
    o_ref[...] = _layer_math(p_ref, x_ref, cnt_ref, wl_ref, wr_ref, b_ref)


def _tc_layer2_body(p_ref, x_ref, cnt_ref, wl_ref, wr_ref, b_ref, batch_ref,
                    g_ref):
    h = _layer_math(p_ref, x_ref, cnt_ref, wl_ref, wr_ref, b_ref)
    oh = (batch_ref[...] ==
          lax.broadcasted_iota(jnp.int32, (BLK, B), 1)).astype(jnp.float32)
    contrib = lax.dot_general(oh, h, (((0,), (0,)), ((), ())),
                              preferred_element_type=jnp.float32,
                              precision=lax.Precision.HIGHEST)

    @pl.when(pl.program_id(0) == 0)
    def _():
        g_ref[...] = jnp.zeros_like(g_ref)
    g_ref[...] += contrib


_COMMON_SPECS = [
    pl.BlockSpec((NC, BLK, D), lambda i: (0, i, 0)),
    pl.BlockSpec((BLK, D), lambda i: (i, 0)),
    pl.BlockSpec((BLK, NC * CW), lambda i: (i, 0)),
    pl.BlockSpec((D, D), lambda i: (0, 0)),
    pl.BlockSpec((D, D), lambda i: (0, 0)),
    pl.BlockSpec((1, D), lambda i: (0, 0)),
]


def _tc_layer(p, x, cnt2, wlT, wrT, b2):
    return pl.pallas_call(
        _tc_layer_body,
        grid=(GRID,),
        in_specs=_COMMON_SPECS,
        out_specs=pl.BlockSpec((BLK, D), lambda i: (i, 0)),
        out_shape=jax.ShapeDtypeStruct((N, D), jnp.float32),
    )(p, x, cnt2, wlT, wrT, b2)


def _tc_layer2(p, x, cnt2, wlT, wrT, b2, batch2):
    return pl.pallas_call(
        _tc_layer2_body,
        grid=(GRID,),
        in_specs=_COMMON_SPECS + [pl.BlockSpec((BLK, 1), lambda i: (i, 0))],
        out_specs=pl.BlockSpec((B, D), lambda i: (0, 0)),
        out_shape=jax.ShapeDtypeStruct((B, D), jnp.float32),
    )(p, x, cnt2, wlT, wrT, b2, batch2)


def kernel(x_raw, edge_index, batch, W_l0, b_l0, W_r0, W_l1, b_l1, W_r1):
    src4 = edge_index[0].reshape(NW, NBLK, IBLK, CHUNK)
    dst4 = edge_index[1].reshape(NW, NBLK, IBLK, CHUNK)
    part0, cntp = _make_sc_agg(True)(x_raw, src4, dst4)
    cnt2 = cntp.T
    h0 = _tc_layer(part0, x_raw, cnt2, W_l0.T, W_r0.T, b_l0.reshape(1, D))
    (part1,) = _make_sc_agg(False)(h0, src4, dst4)
    g = _tc_layer2(part1, h0, cnt2, W_l1.T, W_r1.T, b_l1.reshape(1, D),
                   batch.reshape(N, 1))
    return g


# speedup vs baseline: 11.3463x; 1.0567x over previous
import functools

import jax
import jax.numpy as jnp
from jax import lax
from jax.experimental import pallas as pl
from jax.experimental.pallas import tpu as pltpu
from jax.experimental.pallas import tpu_sc as plsc

N = 10000
E = 320000
D = 128
B = 64

NC = 2
NS = 16
NW = NC * NS
EPT = E // NW
CHUNK = 80
NITER = EPT // CHUNK
IBLK = 25
NBLK = NITER // IBLK
NPAIR = (IBLK - 1) // 2
TROWS = 640
CP = 80
NQ = TROWS // CP
CW = 16

BLK = 1000
GRID = N // BLK


def _mesh():
    return plsc.VectorSubcoreMesh(core_axis_name="c", subcore_axis_name="s",
                                  num_cores=NC, num_subcores=NS)


def _own_copies(sid, r0, fn):
    notlast = sid < NS - 1
    for q in range(NQ):
        if q < 5:
            fn(r0 + q * CP)
        else:
            @pl.when(notlast)
            def _():
                fn(r0 + q * CP)


@functools.lru_cache(maxsize=None)
def _make_sc_agg(with_cnt: bool):
    out_type = [jax.ShapeDtypeStruct((NC, N, D), jnp.float32)]
    scratch = [
        pltpu.VMEM((IBLK, CHUNK), jnp.int32),
        pltpu.VMEM((IBLK, CHUNK), jnp.int32),
        pltpu.VMEM((IBLK, CHUNK), jnp.int32),
        pltpu.VMEM((IBLK, CHUNK), jnp.int32),
        pltpu.VMEM((CHUNK, D), jnp.float32),
        pltpu.VMEM((CHUNK, D), jnp.float32),
        pltpu.VMEM_SHARED((N, D), jnp.float32),
        pltpu.SemaphoreType.DMA,
        pltpu.SemaphoreType.DMA,
        pltpu.SemaphoreType.DMA,
    ]
    if with_cnt:
        out_type.append(jax.ShapeDtypeStruct((NW, N), jnp.float32))
        scratch.append(pltpu.VMEM((N,), jnp.float32))

    def body(*refs):
        if with_cnt:
            (x_hbm, src_hbm, dst_hbm, part_hbm, cntp_hbm,
             src_i0, dst_i0, src_i1, dst_i1, rows0, rows1, acc,
             sem0, sem1, semi, cnt_v) = refs
        else:
            (x_hbm, src_hbm, dst_hbm, part_hbm,
             src_i0, dst_i0, src_i1, dst_i1, rows0, rows1, acc,
             sem0, sem1, semi) = refs
        srcs = [src_i0, src_i1]
        dsts = [dst_i0, dst_i1]

        cid = lax.axis_index("c")
        sid = lax.axis_index("s")
        wid = cid * NS + sid
        r0 = sid * TROWS

        def zrow(i, c):
            for j in range(D // 16):
                rows0[i, pl.ds(j * 16, 16)] = jnp.zeros((16,), jnp.float32)
            return c
        lax.fori_loop(0, CHUNK, zrow, 0)
        _own_copies(sid, r0,
                    lambda rr: pltpu.sync_copy(rows0, acc.at[pl.ds(rr, CP)]))
        if with_cnt:
            def zcnt(i, c):
                cnt_v[pl.ds(i * 16, 16)] = jnp.zeros((16,), jnp.float32)
                return c
            lax.fori_loop(0, N // 16, zcnt, 0)
        ones = jnp.full((16,), 1.0, jnp.float32)

        pltpu.sync_copy(src_hbm.at[wid, 0], srcs[0])
        pltpu.sync_copy(dst_hbm.at[wid, 0], dsts[0])

        plsc.subcore_barrier()

        for b in range(NBLK):
            src_i = srcs[b % 2]
            dst_i = dsts[b % 2]
            if b + 1 < NBLK:
                pltpu.async_copy(src_hbm.at[wid, b + 1], srcs[(b + 1) % 2],
                                 semi)
                pltpu.async_copy(dst_hbm.at[wid, b + 1], dsts[(b + 1) % 2],
                                 semi)

            def fire(it, buf, sem):
                pltpu.async_copy(x_hbm.at[src_i.at[it]], buf, sem)

            def drain(buf, sem):
                pltpu.make_async_copy(x_hbm.at[src_i.at[0]], buf, sem).wait()

            def scat(it, buf):
                if with_cnt:
                    pltpu.sync_copy(buf, acc.at[dst_i.at[it]], add=True)
                if with_cnt:
                    for j in range(CHUNK // 16):
                        idx = dst_i[it, pl.ds(j * 16, 16)]
                        plsc.addupdate_scatter(cnt_v, [idx], ones)

            fire(0, rows0, sem0)

            def step(k, c):
                g = 2 * k
                fire(g + 1, rows1, sem1)
                drain(rows0, sem0)
                scat(g, rows0)
                fire(g + 2, rows0, sem0)
                drain(rows1, sem1)
                scat(g + 1, rows1)
                return c
            lax.fori_loop(0, NPAIR, step, 0)
            drain(rows0, sem0)
            scat(IBLK - 1, rows0)

            if b + 1 < NBLK:
                pltpu.make_async_copy(src_hbm.at[wid, b + 1],
                                      srcs[(b + 1) % 2], semi).wait()
                pltpu.make_async_copy(dst_hbm.at[wid, b + 1],
                                      dsts[(b + 1) % 2], semi).wait()

        plsc.subcore_barrier()

        def cp_out(rr):
            pltpu.sync_copy(acc.at[pl.ds(rr, CP)], rows0)
            pltpu.sync_copy(rows0, part_hbm.at[cid, pl.ds(rr, CP)])
        _own_copies(sid, r0, cp_out)
        if with_cnt:
            pltpu.sync_copy(cnt_v, cntp_hbm.at[wid])

    params = pltpu.CompilerParams(needs_layout_passes=False)
    return pl.kernel(body, out_type=out_type, mesh=_mesh(),
                     compiler_params=params, scratch_types=scratch)


def _layer_math(p_ref, x_ref, cnt_ref, wl_ref, wr_ref, b_ref):
    p = p_ref[...]
    agg = p[0] + p[1]
    cnt = jnp.sum(cnt_ref[...], axis=1, keepdims=True)
    mean = agg / jnp.maximum(cnt, 1.0)
    z = (jnp.dot(mean, wl_ref[...], preferred_element_type=jnp.float32,
                 precision=lax.Precision.HIGHEST)
         + jnp.dot(x_ref[...], wr_ref[...], preferred_element_type=jnp.float32,
                   precision=lax.Precision.HIGHEST)
         + b_ref[...])
    nrm = jnp.sqrt(jnp.sum(z * z, axis=1, keepdims=True))
    h = z / jnp.maximum(nrm, 1e-12)
    return jnp.maximum(h, 0.0)


def _tc_layer_body(p_ref, x_ref, cnt_ref, wl_ref, wr_ref, b_ref, o_ref):
    o_ref[...] = _layer_math(p_ref, x_ref, cnt_ref, wl_ref, wr_ref, b_ref)


def _tc_layer2_body(p_ref, x_ref, cnt_ref, wl_ref, wr_ref, b_ref, batch_ref,
                    g_ref):
    h = _layer_math(p_ref, x_ref, cnt_ref, wl_ref, wr_ref, b_ref)
    oh = (batch_ref[...] ==
          lax.broadcasted_iota(jnp.int32, (BLK, B), 1)).astype(jnp.float32)
    contrib = lax.dot_general(oh, h, (((0,), (0,)), ((), ())),
                              preferred_element_type=jnp.float32,
                              precision=lax.Precision.HIGHEST)

    @pl.when(pl.program_id(0) == 0)
    def _():
        g_ref[...] = jnp.zeros_like(g_ref)
    g_ref[...] += contrib


_COMMON_SPECS = [
    pl.BlockSpec((NC, BLK, D), lambda i: (0, i, 0)),
    pl.BlockSpec((BLK, D), lambda i: (i, 0)),
    pl.BlockSpec((BLK, NC * CW), lambda i: (i, 0)),
    pl.BlockSpec((D, D), lambda i: (0, 0)),
    pl.BlockSpec((D, D), lambda i: (0, 0)),
    pl.BlockSpec((1, D), lambda i: (0, 0)),
]


def _tc_layer(p, x, cnt2, wlT, wrT, b2):
    return pl.pallas_call(
        _tc_layer_body,
        grid=(GRID,),
        in_specs=_COMMON_SPECS,
        out_specs=pl.BlockSpec((BLK, D), lambda i: (i, 0)),
        out_shape=jax.ShapeDtypeStruct((N, D), jnp.float32),
    )(p, x, cnt2, wlT, wrT, b2)


def _tc_layer2(p, x, cnt2, wlT, wrT, b2, batch2):
    return pl.pallas_call(
        _tc_layer2_body,
        grid=(GRID,),
        in_specs=_COMMON_SPECS + [pl.BlockSpec((BLK, 1), lambda i: (i, 0))],
        out_specs=pl.BlockSpec((B, D), lambda i: (0, 0)),
        out_shape=jax.ShapeDtypeStruct((B, D), jnp.float32),
    )(p, x, cnt2, wlT, wrT, b2, batch2)


def kernel(x_raw, edge_index, batch, W_l0, b_l0, W_r0, W_l1, b_l1, W_r1):
    src4 = edge_index[0].reshape(NW, NBLK, IBLK, CHUNK)
    dst4 = edge_index[1].reshape(NW, NBLK, IBLK, CHUNK)
    part0, cntp = _make_sc_agg(True)(x_raw, src4, dst4)
    cnt2 = cntp.T
    h0 = _tc_layer(part0, x_raw, cnt2, W_l0.T, W_r0.T, b_l0.reshape(1, D))
    (part1,) = _make_sc_agg(False)(h0, src4, dst4)
    g = _tc_layer2(part1, h0, cnt2, W_l1.T, W_r1.T, b_l1.reshape(1, D),
                   batch.reshape(N, 1))
    return g


# speedup vs baseline: 11.5550x; 1.0184x over previous
import functools

import jax
import jax.numpy as jnp
from jax import lax
from jax.experimental import pallas as pl
from jax.experimental.pallas import tpu as pltpu
from jax.experimental.pallas import tpu_sc as plsc

N = 10000
E = 320000
D = 128
B = 64

NC = 2
NS = 16
NW = NC * NS
EPT = E // NW
CHUNK = 80
NITER = EPT // CHUNK
IBLK = 25
NBLK = NITER // IBLK
NPAIR = (IBLK - 1) // 2
TROWS = 640
CP = 80
NQ = TROWS // CP
CW = 16

BLK = 1000
GRID = N // BLK


def _mesh():
    return plsc.VectorSubcoreMesh(core_axis_name="c", subcore_axis_name="s",
                                  num_cores=NC, num_subcores=NS)


def _own_copies(sid, r0, fn):
    notlast = sid < NS - 1
    for q in range(NQ):
        if q < 5:
            fn(r0 + q * CP)
        else:
            @pl.when(notlast)
            def _():
                fn(r0 + q * CP)


@functools.lru_cache(maxsize=None)
def _make_sc_agg(with_cnt: bool):
    R = 3 if with_cnt else 4
    out_type = [jax.ShapeDtypeStruct((NC, N, D), jnp.float32)]
    scratch = (
        [pltpu.VMEM((IBLK, CHUNK), jnp.int32)] * 2
        + [pltpu.VMEM((CHUNK, D), jnp.float32)] * R
        + [pltpu.VMEM_SHARED((N, D), jnp.float32)]
        + [pltpu.SemaphoreType.DMA] * (2 * R)
    )
    if with_cnt:
        out_type.append(jax.ShapeDtypeStruct((NW, N), jnp.float32))
        scratch.append(pltpu.VMEM((N,), jnp.float32))

    def body(*refs):
        if with_cnt:
            (x_hbm, src_hbm, dst_hbm, part_hbm, cntp_hbm) = refs[:5]
            rest = refs[5:]
        else:
            (x_hbm, src_hbm, dst_hbm, part_hbm) = refs[:4]
            rest = refs[4:]
        src_i, dst_i = rest[0], rest[1]
        rows = list(rest[2:2 + R])
        acc = rest[2 + R]
        sems_g = list(rest[3 + R:3 + 2 * R])
        sems_s = list(rest[3 + 2 * R:3 + 3 * R])
        cnt_v = rest[3 + 3 * R] if with_cnt else None

        cid = lax.axis_index("c")
        sid = lax.axis_index("s")
        wid = cid * NS + sid
        r0 = sid * TROWS

        def zrow(i, c):
            for j in range(D // 16):
                rows[0][i, pl.ds(j * 16, 16)] = jnp.zeros((16,), jnp.float32)
            return c
        lax.fori_loop(0, CHUNK, zrow, 0)
        _own_copies(sid, r0,
                    lambda rr: pltpu.sync_copy(rows[0], acc.at[pl.ds(rr, CP)]))
        if with_cnt:
            def zcnt(i, c):
                cnt_v[pl.ds(i * 16, 16)] = jnp.zeros((16,), jnp.float32)
                return c
            lax.fori_loop(0, N // 16, zcnt, 0)
        ones = jnp.full((16,), 1.0, jnp.float32)

        plsc.subcore_barrier()

        def fire_g(it, b):
            pltpu.async_copy(x_hbm.at[src_i.at[it]], rows[b], sems_g[b])

        def wait_g(b):
            pltpu.make_async_copy(x_hbm.at[src_i.at[0]], rows[b],
                                  sems_g[b]).wait()

        def fire_s(it, b):
            pltpu.async_copy(rows[b], acc.at[dst_i.at[it]], sems_s[b],
                             add=True)
            if with_cnt:
                for j in range(CHUNK // 16):
                    idx = dst_i[it, pl.ds(j * 16, 16)]
                    plsc.addupdate_scatter(cnt_v, [idx], ones)

        def wait_s(b):
            pltpu.make_async_copy(rows[b], acc.at[dst_i.at[0]],
                                  sems_s[b]).wait()

        K = (IBLK - R) // R
        TAIL = R * K + 1

        for blk in range(NBLK):
            pltpu.sync_copy(src_hbm.at[wid, blk], src_i)
            pltpu.sync_copy(dst_hbm.at[wid, blk], dst_i)

            for r in range(R - 1):
                fire_g(r, r)
            wait_g(0)
            fire_s(0, 0)
            fire_g(R - 1, R - 1)

            def step(k, c):
                for t in range(R):
                    i = R * k + t + 1
                    b = (t + 1) % R
                    wait_g(b)
                    fire_s(i, b)
                    wait_s(t)
                    fire_g(i + R - 1, t)
                return c
            lax.fori_loop(0, K, step, 0)

            for i in range(TAIL, IBLK):
                b = i % R
                wait_g(b)
                fire_s(i, b)
                wait_s((i - 1) % R)
                if i + R - 1 < IBLK:
                    fire_g(i + R - 1, (i - 1) % R)
            wait_s((IBLK - 1) % R)

        plsc.subcore_barrier()

        def cp_out(rr):
            pltpu.sync_copy(acc.at[pl.ds(rr, CP)], rows[0])
            pltpu.sync_copy(rows[0], part_hbm.at[cid, pl.ds(rr, CP)])
        _own_copies(sid, r0, cp_out)
        if with_cnt:
            pltpu.sync_copy(cnt_v, cntp_hbm.at[wid])

    params = pltpu.CompilerParams(needs_layout_passes=False)
    return pl.kernel(body, out_type=out_type, mesh=_mesh(),
                     compiler_params=params, scratch_types=scratch)


def _layer_math(p_ref, x_ref, cnt_ref, wl_ref, wr_ref, b_ref):
    p = p_ref[...]
    agg = p[0] + p[1]
    cnt = jnp.sum(cnt_ref[...], axis=1, keepdims=True)
    mean = agg / jnp.maximum(cnt, 1.0)
    z = (jnp.dot(mean, wl_ref[...], preferred_element_type=jnp.float32,
                 precision=lax.Precision.HIGHEST)
         + jnp.dot(x_ref[...], wr_ref[...], preferred_element_type=jnp.float32,
                   precision=lax.Precision.HIGHEST)
         + b_ref[...])
    nrm = jnp.sqrt(jnp.sum(z * z, axis=1, keepdims=True))
    h = z / jnp.maximum(nrm, 1e-12)
    return jnp.maximum(h, 0.0)


def _tc_layer_body(p_ref, x_ref, cnt_ref, wl_ref, wr_ref, b_ref, o_ref):
    o_ref[...] = _layer_math(p_ref, x_ref, cnt_ref, wl_ref, wr_ref, b_ref)


def _tc_layer2_body(p_ref, x_ref, cnt_ref, wl_ref, wr_ref, b_ref, batch_ref,
                    g_ref):
    h = _layer_math(p_ref, x_ref, cnt_ref, wl_ref, wr_ref, b_ref)
    oh = (batch_ref[...] ==
          lax.broadcasted_iota(jnp.int32, (BLK, B), 1)).astype(jnp.float32)
    contrib = lax.dot_general(oh, h, (((0,), (0,)), ((), ())),
                              preferred_element_type=jnp.float32,
                              precision=lax.Precision.HIGHEST)

    @pl.when(pl.program_id(0) == 0)
    def _():
        g_ref[...] = jnp.zeros_like(g_ref)
    g_ref[...] += contrib


_COMMON_SPECS = [
    pl.BlockSpec((NC, BLK, D), lambda i: (0, i, 0)),
    pl.BlockSpec((BLK, D), lambda i: (i, 0)),
    pl.BlockSpec((BLK, NC * CW), lambda i: (i, 0)),
    pl.BlockSpec((D, D), lambda i: (0, 0)),
    pl.BlockSpec((D, D), lambda i: (0, 0)),
    pl.BlockSpec((1, D), lambda i: (0, 0)),
]


def _tc_layer(p, x, cnt2, wlT, wrT, b2):
    return pl.pallas_call(
        _tc_layer_body,
        grid=(GRID,),
        in_specs=_COMMON_SPECS,
        out_specs=pl.BlockSpec((BLK, D), lambda i: (i, 0)),
        out_shape=jax.ShapeDtypeStruct((N, D), jnp.float32),
    )(p, x, cnt2, wlT, wrT, b2)


def _tc_layer2(p, x, cnt2, wlT, wrT, b2, batch2):
    return pl.pallas_call(
        _tc_layer2_body,
        grid=(GRID,),
        in_specs=_COMMON_SPECS + [pl.BlockSpec((BLK, 1), lambda i: (i, 0))],
        out_specs=pl.BlockSpec((B, D), lambda i: (0, 0)),
        out_shape=jax.ShapeDtypeStruct((B, D), jnp.float32),
    )(p, x, cnt2, wlT, wrT, b2, batch2)


def kernel(x_raw, edge_index, batch, W_l0, b_l0, W_r0, W_l1, b_l1, W_r1):
    src4 = edge_index[0].reshape(NW, NBLK, IBLK, CHUNK)
    dst4 = edge_index[1].reshape(NW, NBLK, IBLK, CHUNK)
    part0, cntp = _make_sc_agg(True)(x_raw, src4, dst4)
    cnt2 = cntp.T
    h0 = _tc_layer(part0, x_raw, cnt2, W_l0.T, W_r0.T, b_l0.reshape(1, D))
    (part1,) = _make_sc_agg(False)(h0, src4, dst4)
    g = _tc_layer2(part1, h0, cnt2, W_l1.T, W_r1.T, b_l1.reshape(1, D),
                   batch.reshape(N, 1))
    return g


# speedup vs baseline: 11.6251x; 1.0061x over previous
import functools

import jax
import jax.numpy as jnp
from jax import lax
from jax.experimental import pallas as pl
from jax.experimental.pallas import tpu as pltpu
from jax.experimental.pallas import tpu_sc as plsc

N = 10000
E = 320000
D = 128
B = 64

NC = 2
NS = 16
NW = NC * NS
EPT = E // NW
CHUNK = 80
NITER = EPT // CHUNK
IBLK = 25
NBLK = NITER // IBLK
NPAIR = (IBLK - 1) // 2
TROWS = 640
CP = 80
NQ = TROWS // CP
CW = 16

BLK = 2000
GRID = N // BLK


def _mesh():
    return plsc.VectorSubcoreMesh(core_axis_name="c", subcore_axis_name="s",
                                  num_cores=NC, num_subcores=NS)


def _own_copies(sid, r0, fn):
    notlast = sid < NS - 1
    for q in range(NQ):
        if q < 5:
            fn(r0 + q * CP)
        else:
            @pl.when(notlast)
            def _():
                fn(r0 + q * CP)


@functools.lru_cache(maxsize=None)
def _make_sc_agg(with_cnt: bool):
    R = 3 if with_cnt else 4
    out_type = [jax.ShapeDtypeStruct((NC, N, D), jnp.float32)]
    scratch = (
        [pltpu.VMEM((IBLK, CHUNK), jnp.int32)] * 2
        + [pltpu.VMEM((CHUNK, D), jnp.float32)] * R
        + [pltpu.VMEM_SHARED((N, D), jnp.float32)]
        + [pltpu.SemaphoreType.DMA] * (2 * R)
    )
    if with_cnt:
        out_type.append(jax.ShapeDtypeStruct((NW, N), jnp.float32))
        scratch.append(pltpu.VMEM((N,), jnp.float32))

    def body(*refs):
        if with_cnt:
            (x_hbm, src_hbm, dst_hbm, zb_hbm, part_hbm, cntp_hbm) = refs[:6]
            rest = refs[6:]
        else:
            (x_hbm, src_hbm, dst_hbm, zb_hbm, part_hbm) = refs[:5]
            rest = refs[5:]
        src_i, dst_i = rest[0], rest[1]
        rows = list(rest[2:2 + R])
        acc = rest[2 + R]
        sems_g = list(rest[3 + R:3 + 2 * R])
        sems_s = list(rest[3 + 2 * R:3 + 3 * R])
        cnt_v = rest[3 + 3 * R] if with_cnt else None

        cid = lax.axis_index("c")
        sid = lax.axis_index("s")
        wid = cid * NS + sid
        r0 = sid * TROWS

        notlast = sid < NS - 1

        @pl.when(notlast)
        def _():
            pltpu.sync_copy(zb_hbm, acc.at[pl.ds(r0, TROWS)])

        @pl.when(jnp.logical_not(notlast))
        def _():
            pltpu.sync_copy(zb_hbm.at[pl.ds(0, N - (NS - 1) * TROWS)],
                            acc.at[pl.ds((NS - 1) * TROWS,
                                         N - (NS - 1) * TROWS)])
        if with_cnt:
            def zcnt(i, c):
                cnt_v[pl.ds(i * 16, 16)] = jnp.zeros((16,), jnp.float32)
                return c
            lax.fori_loop(0, N // 16, zcnt, 0)
        ones = jnp.full((16,), 1.0, jnp.float32)

        plsc.subcore_barrier()

        def fire_g(it, b):
            pltpu.async_copy(x_hbm.at[src_i.at[it]], rows[b], sems_g[b])

        def wait_g(b):
            pltpu.make_async_copy(x_hbm.at[src_i.at[0]], rows[b],
                                  sems_g[b]).wait()

        def fire_s(it, b):
            pltpu.async_copy(rows[b], acc.at[dst_i.at[it]], sems_s[b],
                             add=True)
            if with_cnt:
                for j in range(CHUNK // 16):
                    idx = dst_i[it, pl.ds(j * 16, 16)]
                    plsc.addupdate_scatter(cnt_v, [idx], ones)

        def wait_s(b):
            pltpu.make_async_copy(rows[b], acc.at[dst_i.at[0]],
                                  sems_s[b]).wait()

        K = (IBLK - R) // R
        TAIL = R * K + 1

        for blk in range(NBLK):
            pltpu.sync_copy(src_hbm.at[wid, blk], src_i)
            pltpu.sync_copy(dst_hbm.at[wid, blk], dst_i)

            for r in range(R - 1):
                fire_g(r, r)
            wait_g(0)
            fire_s(0, 0)
            fire_g(R - 1, R - 1)

            def step(k, c):
                for t in range(R):
                    i = R * k + t + 1
                    b = (t + 1) % R
                    wait_g(b)
                    fire_s(i, b)
                    wait_s(t)
                    fire_g(i + R - 1, t)
                return c
            lax.fori_loop(0, K, step, 0)

            for i in range(TAIL, IBLK):
                b = i % R
                wait_g(b)
                fire_s(i, b)
                wait_s((i - 1) % R)
                if i + R - 1 < IBLK:
                    fire_g(i + R - 1, (i - 1) % R)
            wait_s((IBLK - 1) % R)

        plsc.subcore_barrier()

        @pl.when(notlast)
        def _():
            pltpu.sync_copy(acc.at[pl.ds(r0, TROWS)],
                            part_hbm.at[cid, pl.ds(r0, TROWS)])

        @pl.when(jnp.logical_not(notlast))
        def _():
            LR = N - (NS - 1) * TROWS
            pltpu.sync_copy(acc.at[pl.ds((NS - 1) * TROWS, LR)],
                            part_hbm.at[cid, pl.ds((NS - 1) * TROWS, LR)])
        if with_cnt:
            pltpu.sync_copy(cnt_v, cntp_hbm.at[wid])

    params = pltpu.CompilerParams(needs_layout_passes=False)
    return pl.kernel(body, out_type=out_type, mesh=_mesh(),
                     compiler_params=params, scratch_types=scratch)


def _layer_math(p_ref, x_ref, cnt_ref, wl_ref, wr_ref, b_ref):
    p = p_ref[...]
    agg = p[0] + p[1]
    cnt = jnp.sum(cnt_ref[...], axis=1, keepdims=True)
    mean = agg / jnp.maximum(cnt, 1.0)
    z = (jnp.dot(mean, wl_ref[...], preferred_element_type=jnp.float32,
                 precision=lax.Precision.HIGHEST)
         + jnp.dot(x_ref[...], wr_ref[...], preferred_element_type=jnp.float32,
                   precision=lax.Precision.HIGHEST)
         + b_ref[...])
    nrm = jnp.sqrt(jnp.sum(z * z, axis=1, keepdims=True))
    h = z / jnp.maximum(nrm, 1e-12)
    return jnp.maximum(h, 0.0)


def _tc_layer_body(p_ref, x_ref, cnt_ref, wl_ref, wr_ref, b_ref, o_ref):
    o_ref[...] = _layer_math(p_ref, x_ref, cnt_ref, wl_ref, wr_ref, b_ref)


def _tc_layer2_body(p_ref, x_ref, cnt_ref, wl_ref, wr_ref, b_ref, batch_ref,
                    g_ref):
    h = _layer_math(p_ref, x_ref, cnt_ref, wl_ref, wr_ref, b_ref)
    oh = (batch_ref[...] ==
          lax.broadcasted_iota(jnp.int32, (BLK, B), 1)).astype(jnp.float32)
    contrib = lax.dot_general(oh, h, (((0,), (0,)), ((), ())),
                              preferred_element_type=jnp.float32,
                              precision=lax.Precision.HIGHEST)

    @pl.when(pl.program_id(0) == 0)
    def _():
        g_ref[...] = jnp.zeros_like(g_ref)
    g_ref[...] += contrib


_COMMON_SPECS = [
    pl.BlockSpec((NC, BLK, D), lambda i: (0, i, 0)),
    pl.BlockSpec((BLK, D), lambda i: (i, 0)),
    pl.BlockSpec((BLK, NC * CW), lambda i: (i, 0)),
    pl.BlockSpec((D, D), lambda i: (0, 0)),
    pl.BlockSpec((D, D), lambda i: (0, 0)),
    pl.BlockSpec((1, D), lambda i: (0, 0)),
]


def _tc_layer(p, x, cnt2, wlT, wrT, b2):
    return pl.pallas_call(
        _tc_layer_body,
        grid=(GRID,),
        in_specs=_COMMON_SPECS,
        out_specs=pl.BlockSpec((BLK, D), lambda i: (i, 0)),
        out_shape=jax.ShapeDtypeStruct((N, D), jnp.float32),
    )(p, x, cnt2, wlT, wrT, b2)


def _tc_layer2(p, x, cnt2, wlT, wrT, b2, batch2):
    return pl.pallas_call(
        _tc_layer2_body,
        grid=(GRID,),
        in_specs=_COMMON_SPECS + [pl.BlockSpec((BLK, 1), lambda i: (i, 0))],
        out_specs=pl.BlockSpec((B, D), lambda i: (0, 0)),
        out_shape=jax.ShapeDtypeStruct((B, D), jnp.float32),
    )(p, x, cnt2, wlT, wrT, b2, batch2)


def kernel(x_raw, edge_index, batch, W_l0, b_l0, W_r0, W_l1, b_l1, W_r1):
    src4 = edge_index[0].reshape(NW, NBLK, IBLK, CHUNK)
    dst4 = edge_index[1].reshape(NW, NBLK, IBLK, CHUNK)
    zb = jnp.zeros((TROWS, D), jnp.float32)
    part0, cntp = _make_sc_agg(True)(x_raw, src4, dst4, zb)
    cnt2 = cntp.T
    h0 = _tc_layer(part0, x_raw, cnt2, W_l0.T, W_r0.T, b_l0.reshape(1, D))
    (part1,) = _make_sc_agg(False)(h0, src4, dst4, zb)
    g = _tc_layer2(part1, h0, cnt2, W_l1.T, W_r1.T, b_l1.reshape(1, D),
                   batch.reshape(N, 1))
    return g


# speedup vs baseline: 11.9055x; 1.0241x over previous
import functools

import jax
import jax.numpy as jnp
from jax import lax
from jax.experimental import pallas as pl
from jax.experimental.pallas import tpu as pltpu
from jax.experimental.pallas import tpu_sc as plsc

N = 10000
E = 320000
D = 128
B = 64

NC = 2
NS = 16
NW = NC * NS
EPT = E // NW
CHUNK = 80
NITER = EPT // CHUNK
IBLK = 25
NBLK = NITER // IBLK
NPAIR = (IBLK - 1) // 2
TROWS = 640
CP = 80
NQ = TROWS // CP
CW = 16

BLK = 2000
GRID = N // BLK


def _mesh():
    return plsc.VectorSubcoreMesh(core_axis_name="c", subcore_axis_name="s",
                                  num_cores=NC, num_subcores=NS)


def _own_copies(sid, r0, fn):
    notlast = sid < NS - 1
    for q in range(NQ):
        if q < 5:
            fn(r0 + q * CP)
        else:
            @pl.when(notlast)
            def _():
                fn(r0 + q * CP)


@functools.lru_cache(maxsize=None)
def _make_sc_agg(with_cnt: bool):
    R = 3
    NIP = 1 if with_cnt else 2
    out_type = [jax.ShapeDtypeStruct((NC, N, D), jnp.float32)]
    scratch = (
        [pltpu.VMEM((IBLK, CHUNK), jnp.int32)] * (2 * NIP)
        + [pltpu.VMEM((CHUNK, D), jnp.float32)] * R
        + [pltpu.VMEM_SHARED((N, D), jnp.float32)]
        + [pltpu.SemaphoreType.DMA] * (2 * R + 1)
    )
    if with_cnt:
        out_type.append(jax.ShapeDtypeStruct((NW, N), jnp.float32))
        scratch.append(pltpu.VMEM((N,), jnp.float32))

    def body(*refs):
        if with_cnt:
            (x_hbm, src_hbm, dst_hbm, zb_hbm, part_hbm, cntp_hbm) = refs[:6]
            rest = refs[6:]
        else:
            (x_hbm, src_hbm, dst_hbm, zb_hbm, part_hbm) = refs[:5]
            rest = refs[5:]
        if NIP == 2:
            src_p = [rest[0], rest[2]]
            dst_p = [rest[1], rest[3]]
        else:
            src_p = [rest[0], rest[0]]
            dst_p = [rest[1], rest[1]]
        o = 2 * NIP
        rows = list(rest[o:o + R])
        acc = rest[o + R]
        sems_g = list(rest[o + 1 + R:o + 1 + 2 * R])
        sems_s = list(rest[o + 1 + 2 * R:o + 1 + 3 * R])
        semi = rest[o + 1 + 3 * R]
        cnt_v = rest[o + 2 + 3 * R] if with_cnt else None

        cid = lax.axis_index("c")
        sid = lax.axis_index("s")
        wid = cid * NS + sid
        r0 = sid * TROWS

        notlast = sid < NS - 1

        @pl.when(notlast)
        def _():
            pltpu.sync_copy(zb_hbm, acc.at[pl.ds(r0, TROWS)])

        @pl.when(jnp.logical_not(notlast))
        def _():
            pltpu.sync_copy(zb_hbm.at[pl.ds(0, N - (NS - 1) * TROWS)],
                            acc.at[pl.ds((NS - 1) * TROWS,
                                         N - (NS - 1) * TROWS)])
        if with_cnt:
            def zcnt(i, c):
                cnt_v[pl.ds(i * 16, 16)] = jnp.zeros((16,), jnp.float32)
                return c
            lax.fori_loop(0, N // 16, zcnt, 0)
        ones = jnp.full((16,), 1.0, jnp.float32)

        plsc.subcore_barrier()

        K = (IBLK - R) // R
        TAIL = R * K + 1

        pltpu.sync_copy(src_hbm.at[wid, 0], src_p[0])
        pltpu.sync_copy(dst_hbm.at[wid, 0], dst_p[0])

        for blk in range(NBLK):
            src_i = src_p[blk % 2]
            dst_i = dst_p[blk % 2]
            if NIP == 2 and blk + 1 < NBLK:
                pltpu.async_copy(src_hbm.at[wid, blk + 1],
                                 src_p[(blk + 1) % 2], semi)
                pltpu.async_copy(dst_hbm.at[wid, blk + 1],
                                 dst_p[(blk + 1) % 2], semi)

            def fire_g(it, b, src_i=src_i):
                pltpu.async_copy(x_hbm.at[src_i.at[it]], rows[b], sems_g[b])

            def wait_g(b, src_i=src_i):
                pltpu.make_async_copy(x_hbm.at[src_i.at[0]], rows[b],
                                      sems_g[b]).wait()

            def fire_s(it, b, dst_i=dst_i):
                pltpu.async_copy(rows[b], acc.at[dst_i.at[it]], sems_s[b],
                                 add=True)
                if with_cnt:
                    for j in range(CHUNK // 16):
                        idx = dst_i[it, pl.ds(j * 16, 16)]
                        plsc.addupdate_scatter(cnt_v, [idx], ones)

            def wait_s(b, dst_i=dst_i):
                pltpu.make_async_copy(rows[b], acc.at[dst_i.at[0]],
                                      sems_s[b]).wait()

            for r in range(R - 1):
                fire_g(r, r)
            wait_g(0)
            fire_s(0, 0)
            fire_g(R - 1, R - 1)

            def step(k, c):
                for t in range(R):
                    i = R * k + t + 1
                    b = (t + 1) % R
                    wait_g(b)
                    fire_s(i, b)
                    wait_s(t)
                    fire_g(i + R - 1, t)
                return c
            lax.fori_loop(0, K, step, 0)

            for i in range(TAIL, IBLK):
                b = i % R
                wait_g(b)
                fire_s(i, b)
                wait_s((i - 1) % R)
                if i + R - 1 < IBLK:
                    fire_g(i + R - 1, (i - 1) % R)
            wait_s((IBLK - 1) % R)

            if NIP == 2 and blk + 1 < NBLK:
                pltpu.make_async_copy(src_hbm.at[wid, blk + 1],
                                      src_p[(blk + 1) % 2], semi).wait()
                pltpu.make_async_copy(dst_hbm.at[wid, blk + 1],
                                      dst_p[(blk + 1) % 2], semi).wait()
            elif NIP == 1 and blk + 1 < NBLK:
                pltpu.sync_copy(src_hbm.at[wid, blk + 1], src_p[0])
                pltpu.sync_copy(dst_hbm.at[wid, blk + 1], dst_p[0])

        plsc.subcore_barrier()

        @pl.when(notlast)
        def _():
            pltpu.sync_copy(acc.at[pl.ds(r0, TROWS)],
                            part_hbm.at[cid, pl.ds(r0, TROWS)])

        @pl.when(jnp.logical_not(notlast))
        def _():
            LR = N - (NS - 1) * TROWS
            pltpu.sync_copy(acc.at[pl.ds((NS - 1) * TROWS, LR)],
                            part_hbm.at[cid, pl.ds((NS - 1) * TROWS, LR)])
        if with_cnt:
            pltpu.sync_copy(cnt_v, cntp_hbm.at[wid])

    params = pltpu.CompilerParams(needs_layout_passes=False)
    return pl.kernel(body, out_type=out_type, mesh=_mesh(),
                     compiler_params=params, scratch_types=scratch)


def _layer_math(p_ref, x_ref, cnt_ref, wl_ref, wr_ref, b_ref):
    p = p_ref[...]
    agg = p[0] + p[1]
    cnt = jnp.sum(cnt_ref[...], axis=1, keepdims=True)
    mean = agg / jnp.maximum(cnt, 1.0)
    z = (jnp.dot(mean, wl_ref[...], preferred_element_type=jnp.float32,
                 precision=lax.Precision.HIGHEST)
         + jnp.dot(x_ref[...], wr_ref[...], preferred_element_type=jnp.float32,
                   precision=lax.Precision.HIGHEST)
         + b_ref[...])
    nrm = jnp.sqrt(jnp.sum(z * z, axis=1, keepdims=True))
    h = z / jnp.maximum(nrm, 1e-12)
    return jnp.maximum(h, 0.0)


def _tc_layer_body(p_ref, x_ref, cnt_ref, wl_ref, wr_ref, b_ref, o_ref):
    o_ref[...] = _layer_math(p_ref, x_ref, cnt_ref, wl_ref, wr_ref, b_ref)


def _tc_layer2_body(p_ref, x_ref, cnt_ref, wl_ref, wr_ref, b_ref, batch_ref,
                    g_ref):
    h = _layer_math(p_ref, x_ref, cnt_ref, wl_ref, wr_ref, b_ref)
    oh = (batch_ref[...] ==
          lax.broadcasted_iota(jnp.int32, (BLK, B), 1)).astype(jnp.float32)
    contrib = lax.dot_general(oh, h, (((0,), (0,)), ((), ())),
                              preferred_element_type=jnp.float32,
                              precision=lax.Precision.HIGHEST)

    @pl.when(pl.program_id(0) == 0)
    def _():
        g_ref[...] = jnp.zeros_like(g_ref)
    g_ref[...] += contrib


_COMMON_SPECS = [
    pl.BlockSpec((NC, BLK, D), lambda i: (0, i, 0)),
    pl.BlockSpec((BLK, D), lambda i: (i, 0)),
    pl.BlockSpec((BLK, NC * CW), lambda i: (i, 0)),
    pl.BlockSpec((D, D), lambda i: (0, 0)),
    pl.BlockSpec((D, D), lambda i: (0, 0)),
    pl.BlockSpec((1, D), lambda i: (0, 0)),
]


def _tc_layer(p, x, cnt2, wlT, wrT, b2):
    return pl.pallas_call(
        _tc_layer_body,
        grid=(GRID,),
        in_specs=_COMMON_SPECS,
        out_specs=pl.BlockSpec((BLK, D), lambda i: (i, 0)),
        out_shape=jax.ShapeDtypeStruct((N, D), jnp.float32),
    )(p, x, cnt2, wlT, wrT, b2)


def _tc_layer2(p, x, cnt2, wlT, wrT, b2, batch2):
    return pl.pallas_call(
        _tc_layer2_body,
        grid=(GRID,),
        in_specs=_COMMON_SPECS + [pl.BlockSpec((BLK, 1), lambda i: (i, 0))],
        out_specs=pl.BlockSpec((B, D), lambda i: (0, 0)),
        out_shape=jax.ShapeDtypeStruct((B, D), jnp.float32),
    )(p, x, cnt2, wlT, wrT, b2, batch2)


def kernel(x_raw, edge_index, batch, W_l0, b_l0, W_r0, W_l1, b_l1, W_r1):
    src4 = edge_index[0].reshape(NW, NBLK, IBLK, CHUNK)
    dst4 = edge_index[1].reshape(NW, NBLK, IBLK, CHUNK)
    zb = jnp.zeros((TROWS, D), jnp.float32)
    part0, cntp = _make_sc_agg(True)(x_raw, src4, dst4, zb)
    cnt2 = cntp.T
    h0 = _tc_layer(part0, x_raw, cnt2, W_l0.T, W_r0.T, b_l0.reshape(1, D))
    (part1,) = _make_sc_agg(False)(h0, src4, dst4, zb)
    g = _tc_layer2(part1, h0, cnt2, W_l1.T, W_r1.T, b_l1.reshape(1, D),
                   batch.reshape(N, 1))
    return g


# speedup vs baseline: 11.9146x; 1.0008x over previous
import functools

import jax
import jax.numpy as jnp
from jax import lax
from jax.experimental import pallas as pl
from jax.experimental.pallas import tpu as pltpu
from jax.experimental.pallas import tpu_sc as plsc

N = 10000
E = 320000
D = 128
B = 64

NC = 2
NS = 16
NW = NC * NS
EPT = E // NW
CHUNK = 80
NITER = EPT // CHUNK
IBLK = 25
NBLK = NITER // IBLK
TROWS = 640

BLK = 2000
GRID = N // BLK


def _mesh():
    return plsc.VectorSubcoreMesh(core_axis_name="c", subcore_axis_name="s",
                                  num_cores=NC, num_subcores=NS)


@functools.lru_cache(maxsize=None)
def _make_sc_agg(with_cnt: bool):
    R = 3
    NIP = 1 if with_cnt else 2
    out_type = [jax.ShapeDtypeStruct((NC, N, D), jnp.float32)]
    scratch = (
        [pltpu.VMEM((IBLK, CHUNK), jnp.int32)] * (2 * NIP)
        + [pltpu.VMEM((CHUNK, D), jnp.float32)] * R
        + [pltpu.VMEM_SHARED((N, D), jnp.float32)]
        + [pltpu.SemaphoreType.DMA] * (2 * R + 1)
    )
    if with_cnt:
        out_type.append(jax.ShapeDtypeStruct((NW, N), jnp.float32))
        scratch.append(pltpu.VMEM((N,), jnp.float32))

    def body(*refs):
        if with_cnt:
            (x_hbm, src_hbm, dst_hbm, zb_hbm, part_hbm, cntp_hbm) = refs[:6]
            rest = refs[6:]
        else:
            (x_hbm, src_hbm, dst_hbm, zb_hbm, part_hbm) = refs[:5]
            rest = refs[5:]
        if NIP == 2:
            src_p = [rest[0], rest[2]]
            dst_p = [rest[1], rest[3]]
        else:
            src_p = [rest[0], rest[0]]
            dst_p = [rest[1], rest[1]]
        o = 2 * NIP
        rows = list(rest[o:o + R])
        acc = rest[o + R]
        sems_g = list(rest[o + 1 + R:o + 1 + 2 * R])
        sems_s = list(rest[o + 1 + 2 * R:o + 1 + 3 * R])
        semi = rest[o + 1 + 3 * R]
        cnt_v = rest[o + 2 + 3 * R] if with_cnt else None

        cid = lax.axis_index("c")
        sid = lax.axis_index("s")
        wid = cid * NS + sid
        r0 = sid * TROWS

        notlast = sid < NS - 1

        @pl.when(notlast)
        def _():
            pltpu.sync_copy(zb_hbm, acc.at[pl.ds(r0, TROWS)])

        @pl.when(jnp.logical_not(notlast))
        def _():
            pltpu.sync_copy(zb_hbm.at[pl.ds(0, N - (NS - 1) * TROWS)],
                            acc.at[pl.ds((NS - 1) * TROWS,
                                         N - (NS - 1) * TROWS)])
        if with_cnt:
            def zcnt(i, c):
                cnt_v[pl.ds(i * 16, 16)] = jnp.zeros((16,), jnp.float32)
                return c
            lax.fori_loop(0, N // 16, zcnt, 0)
        ones = jnp.full((16,), 1.0, jnp.float32)

        plsc.subcore_barrier()

        K = (IBLK - R) // R
        TAIL = R * K + 1

        pltpu.sync_copy(src_hbm.at[wid, 0], src_p[0])
        pltpu.sync_copy(dst_hbm.at[wid, 0], dst_p[0])

        for blk in range(NBLK):
            src_i = src_p[blk % 2]
            dst_i = dst_p[blk % 2]
            if NIP == 2 and blk + 1 < NBLK:
                pltpu.async_copy(src_hbm.at[wid, blk + 1],
                                 src_p[(blk + 1) % 2], semi)
                pltpu.async_copy(dst_hbm.at[wid, blk + 1],
                                 dst_p[(blk + 1) % 2], semi)

            def fire_g(it, b, src_i=src_i):
                pltpu.async_copy(x_hbm.at[src_i.at[it]], rows[b], sems_g[b])

            def wait_g(b, src_i=src_i):
                pltpu.make_async_copy(x_hbm.at[src_i.at[0]], rows[b],
                                      sems_g[b]).wait()

            def fire_s(it, b, dst_i=dst_i):
                pltpu.async_copy(rows[b], acc.at[dst_i.at[it]], sems_s[b],
                                 add=True)
                if with_cnt:
                    for j in range(CHUNK // 16):
                        idx = dst_i[it, pl.ds(j * 16, 16)]
                        plsc.addupdate_scatter(cnt_v, [idx], ones)

            def wait_s(b, dst_i=dst_i):
                pltpu.make_async_copy(rows[b], acc.at[dst_i.at[0]],
                                      sems_s[b]).wait()

            for r in range(R - 1):
                fire_g(r, r)
            wait_g(0)
            fire_s(0, 0)
            fire_g(R - 1, R - 1)

            def step(k, c):
                for t in range(R):
                    i = R * k + t + 1
                    b = (t + 1) % R
                    wait_g(b)
                    fire_s(i, b)
                    wait_s(t)
                    fire_g(i + R - 1, t)
                return c
            lax.fori_loop(0, K, step, 0)

            for i in range(TAIL, IBLK):
                b = i % R
                wait_g(b)
                fire_s(i, b)
                wait_s((i - 1) % R)
                if i + R - 1 < IBLK:
                    fire_g(i + R - 1, (i - 1) % R)
            wait_s((IBLK - 1) % R)

            if NIP == 2 and blk + 1 < NBLK:
                pltpu.make_async_copy(src_hbm.at[wid, blk + 1],
                                      src_p[(blk + 1) % 2], semi).wait()
                pltpu.make_async_copy(dst_hbm.at[wid, blk + 1],
                                      dst_p[(blk + 1) % 2], semi).wait()
            elif NIP == 1 and blk + 1 < NBLK:
                pltpu.sync_copy(src_hbm.at[wid, blk + 1], src_p[0])
                pltpu.sync_copy(dst_hbm.at[wid, blk + 1], dst_p[0])

        plsc.subcore_barrier()

        @pl.when(notlast)
        def _():
            pltpu.sync_copy(acc.at[pl.ds(r0, TROWS)],
                            part_hbm.at[cid, pl.ds(r0, TROWS)])

        @pl.when(jnp.logical_not(notlast))
        def _():
            LR = N - (NS - 1) * TROWS
            pltpu.sync_copy(acc.at[pl.ds((NS - 1) * TROWS, LR)],
                            part_hbm.at[cid, pl.ds((NS - 1) * TROWS, LR)])
        if with_cnt:
            pltpu.sync_copy(cnt_v, cntp_hbm.at[wid])

    params = pltpu.CompilerParams(needs_layout_passes=False)
    return pl.kernel(body, out_type=out_type, mesh=_mesh(),
                     compiler_params=params, scratch_types=scratch)


def _layer_math(p_ref, x_ref, cnt_ref, wl_ref, wr_ref, b_ref):
    p = p_ref[...]
    agg = p[0] + p[1]
    cnt = jnp.sum(cnt_ref[...], axis=1, keepdims=True)
    mean = agg / jnp.maximum(cnt, 1.0)
    z = (jnp.dot(mean, wl_ref[...], preferred_element_type=jnp.float32,
                 precision=lax.Precision.HIGHEST)
         + jnp.dot(x_ref[...], wr_ref[...], preferred_element_type=jnp.float32,
                   precision=lax.Precision.HIGHEST)
         + b_ref[...])
    nrm = jnp.sqrt(jnp.sum(z * z, axis=1, keepdims=True))
    h = z / jnp.maximum(nrm, 1e-12)
    return jnp.maximum(h, 0.0)


def _tc_layer_body(p_ref, x_ref, cnt_ref, wl_ref, wr_ref, b_ref, o_ref):
    o_ref[...] = _layer_math(p_ref, x_ref, cnt_ref, wl_ref, wr_ref, b_ref)


def _tc_layer2_body(p_ref, x_ref, cnt_ref, wl_ref, wr_ref, b_ref, batch_ref,
                    g_ref):
    h = _layer_math(p_ref, x_ref, cnt_ref, wl_ref, wr_ref, b_ref)
    oh = (batch_ref[...] ==
          lax.broadcasted_iota(jnp.int32, (BLK, B), 1)).astype(jnp.float32)
    contrib = lax.dot_general(oh, h, (((0,), (0,)), ((), ())),
                              preferred_element_type=jnp.float32,
                              precision=lax.Precision.HIGHEST)

    @pl.when(pl.program_id(0) == 0)
    def _():
        g_ref[...] = jnp.zeros_like(g_ref)
    g_ref[...] += contrib


_COMMON_SPECS = [
    pl.BlockSpec((NC, BLK, D), lambda i: (0, i, 0)),
    pl.BlockSpec((BLK, D), lambda i: (i, 0)),
    pl.BlockSpec((BLK, NW), lambda i: (i, 0)),
    pl.BlockSpec((D, D), lambda i: (0, 0)),
    pl.BlockSpec((D, D), lambda i: (0, 0)),
    pl.BlockSpec((1, D), lambda i: (0, 0)),
]


def _tc_layer(p, x, cnt2, wlT, wrT, b2):
    return pl.pallas_call(
        _tc_layer_body,
        grid=(GRID,),
        in_specs=_COMMON_SPECS,
        out_specs=pl.BlockSpec((BLK, D), lambda i: (i, 0)),
        out_shape=jax.ShapeDtypeStruct((N, D), jnp.float32),
    )(p, x, cnt2, wlT, wrT, b2)


def _tc_layer2(p, x, cnt2, wlT, wrT, b2, batch2):
    return pl.pallas_call(
        _tc_layer2_body,
        grid=(GRID,),
        in_specs=_COMMON_SPECS + [pl.BlockSpec((BLK, 1), lambda i: (i, 0))],
        out_specs=pl.BlockSpec((B, D), lambda i: (0, 0)),
        out_shape=jax.ShapeDtypeStruct((B, D), jnp.float32),
    )(p, x, cnt2, wlT, wrT, b2, batch2)


def kernel(x_raw, edge_index, batch, W_l0, b_l0, W_r0, W_l1, b_l1, W_r1):
    src4 = edge_index[0].reshape(NW, NBLK, IBLK, CHUNK)
    dst4 = edge_index[1].reshape(NW, NBLK, IBLK, CHUNK)
    zb = jnp.zeros((TROWS, D), jnp.float32)
    part0, cntp = _make_sc_agg(True)(x_raw, src4, dst4, zb)
    cnt2 = cntp.T
    h0 = _tc_layer(part0, x_raw, cnt2, W_l0.T, W_r0.T, b_l0.reshape(1, D))
    (part1,) = _make_sc_agg(False)(h0, src4, dst4, zb)
    g = _tc_layer2(part1, h0, cnt2, W_l1.T, W_r1.T, b_l1.reshape(1, D),
                   batch.reshape(N, 1))
    return g
